# baseline (speedup 1.0000x reference)
import functools

import jax
import jax.numpy as jnp
from jax import lax
from jax.experimental import pallas as pl
from jax.experimental.pallas import tpu as pltpu
from jax.experimental.pallas import tpu_sc as plsc

_B = 16384
_L = 20
_MID = 128
_NC = 2
_NS = 16
_NW = _NC * _NS
_BPW = _B // _NW
_GRP = _BPW // 16

_T1P, _T2P, _T3P = 128, 3072, 1024


def _prep_body(ac_ref, play_ref, atk_ref, dfd_ref, evo_ref, states_ref,
               emb1_ref, emb2_ref, emb3_ref, w1_ref, b1_ref, w2_ref, b2_ref,
               packed_ref, base_ref, t1_ref, t2_ref, t3a_ref, t3d_ref,
               t3e_ref):
    hi = lax.Precision.HIGHEST
    rr = lax.broadcasted_iota(jnp.int32, (128, _L), 0)
    cc = lax.broadcasted_iota(jnp.int32, (128, _L), 1)
    acc = jnp.zeros((128, _BPW), jnp.float32)
    for k, ref in enumerate((ac_ref, play_ref, atk_ref, dfd_ref, evo_ref)):
        q = (rr == cc + k * _L).astype(jnp.float32)
        acc = acc + lax.dot_general(
            q, ref[...].astype(jnp.float32), (((1,), (1,)), ((), ())),
            precision=hi)
    packed_ref[0] = acc.astype(jnp.int32)

    @pl.when(pl.program_id(0) == 0)
    def _():
        w2 = w2_ref[...]
        v = lax.dot_general(w2, w1_ref[...], (((1,), (0,)), ((), ())),
                            precision=hi)
        c = jnp.sum(b1_ref[...] * w2) + b2_ref[0, 0]

        def projT(vk, emb, pad):
            row = lax.dot_general(vk, emb, (((1,), (1,)), ((), ())),
                                  precision=hi)
            if pad:
                row = jnp.concatenate(
                    [row, jnp.zeros((1, pad), jnp.float32)], axis=1)
            return row

        base_ref[...] = projT(v[:, 0:128], states_ref[...], 0) + c
        t1_ref[...] = projT(v[:, 128:256], emb1_ref[...], _T1P - 5)
        t2_ref[...] = projT(v[:, 256:384], emb2_ref[...], _T2P - 3000)
        t3a_ref[...] = projT(v[:, 384:512], emb3_ref[...], _T3P - 1000)
        t3d_ref[...] = projT(v[:, 512:640], emb3_ref[...], _T3P - 1000)
        t3e_ref[...] = projT(v[:, 640:768], emb3_ref[...], _T3P - 1000)


_prep = pl.pallas_call(
    _prep_body,
    grid=(_NW,),
    in_specs=[
        pl.BlockSpec((_BPW, _L), lambda i: (i, 0)),
        pl.BlockSpec((_BPW, _L), lambda i: (i, 0)),
        pl.BlockSpec((_BPW, _L), lambda i: (i, 0)),
        pl.BlockSpec((_BPW, _L), lambda i: (i, 0)),
        pl.BlockSpec((_BPW, _L), lambda i: (i, 0)),
        pl.BlockSpec((_B, 128), lambda i: (0, 0)),
        pl.BlockSpec((5, 128), lambda i: (0, 0)),
        pl.BlockSpec((3000, 128), lambda i: (0, 0)),
        pl.BlockSpec((1000, 128), lambda i: (0, 0)),
        pl.BlockSpec((128, 768), lambda i: (0, 0)),
        pl.BlockSpec((1, 128), lambda i: (0, 0)),
        pl.BlockSpec((1, 128), lambda i: (0, 0)),
        pl.BlockSpec((1, 1), lambda i: (0, 0)),
    ],
    out_specs=[
        pl.BlockSpec((1, 128, _BPW), lambda i: (i, 0, 0)),
        pl.BlockSpec((1, _B), lambda i: (0, 0)),
        pl.BlockSpec((1, _T1P), lambda i: (0, 0)),
        pl.BlockSpec((1, _T2P), lambda i: (0, 0)),
        pl.BlockSpec((1, _T3P), lambda i: (0, 0)),
        pl.BlockSpec((1, _T3P), lambda i: (0, 0)),
        pl.BlockSpec((1, _T3P), lambda i: (0, 0)),
    ],
    out_shape=[
        jax.ShapeDtypeStruct((_NW, 128, _BPW), jnp.int32),
        jax.ShapeDtypeStruct((1, _B), jnp.float32),
        jax.ShapeDtypeStruct((1, _T1P), jnp.float32),
        jax.ShapeDtypeStruct((1, _T2P), jnp.float32),
        jax.ShapeDtypeStruct((1, _T3P), jnp.float32),
        jax.ShapeDtypeStruct((1, _T3P), jnp.float32),
        jax.ShapeDtypeStruct((1, _T3P), jnp.float32),
    ],
)


def _make_sc_gather():
    mesh = plsc.VectorSubcoreMesh(core_axis_name="c", subcore_axis_name="s")

    @functools.partial(
        pl.kernel,
        mesh=mesh,
        out_type=jax.ShapeDtypeStruct((_B,), jnp.float32),
        compiler_params=pltpu.CompilerParams(needs_layout_passes=False),
        scratch_types=[
            pltpu.VMEM((128, _BPW), jnp.int32),
            pltpu.VMEM((_T1P,), jnp.float32),
            pltpu.VMEM((_T2P,), jnp.float32),
            pltpu.VMEM((_T3P,), jnp.float32),
            pltpu.VMEM((_T3P,), jnp.float32),
            pltpu.VMEM((_T3P,), jnp.float32),
            pltpu.VMEM((_BPW,), jnp.float32),
            pltpu.VMEM((_BPW,), jnp.float32),
        ],
    )
    def sc_k(idx_hbm, base_hbm, t1_hbm, t2_hbm, t3a_hbm, t3d_hbm, t3e_hbm,
             out_hbm,
             tw, t1_v, t2_v, t3a_v, t3d_v, t3e_v, base_v, out_v):
        wid = lax.axis_index("s") * _NC + lax.axis_index("c")
        b0 = wid * _BPW
        pltpu.sync_copy(idx_hbm.at[wid], tw)
        pltpu.sync_copy(t1_hbm, t1_v)
        pltpu.sync_copy(t2_hbm, t2_v)
        pltpu.sync_copy(t3a_hbm, t3a_v)
        pltpu.sync_copy(t3d_hbm, t3d_v)
        pltpu.sync_copy(t3e_hbm, t3e_v)
        pltpu.sync_copy(base_hbm.at[pl.ds(b0, _BPW)], base_v)

        def group(gi, carry):
            s0 = gi * 16
            a1 = base_v[pl.ds(s0, 16)]
            a2 = a1 - a1
            a3 = a2
            a4 = a2
            a5 = a2
            for l in range(_L):
                a1 = a1 + plsc.load_gather(t1_v, [tw[l, pl.ds(s0, 16)]])
                a2 = a2 + plsc.load_gather(t2_v, [tw[_L + l, pl.ds(s0, 16)]])
                a3 = a3 + plsc.load_gather(
                    t3a_v, [tw[2 * _L + l, pl.ds(s0, 16)]])
                a4 = a4 + plsc.load_gather(
                    t3d_v, [tw[3 * _L + l, pl.ds(s0, 16)]])
                a5 = a5 + plsc.load_gather(
                    t3e_v, [tw[4 * _L + l, pl.ds(s0, 16)]])
            out_v[pl.ds(s0, 16)] = (a1 + a2) + (a3 + a4) + a5
            return carry

        lax.fori_loop(0, _GRP, group, 0)
        pltpu.sync_copy(out_v, out_hbm.at[pl.ds(b0, _BPW)])

    return sc_k


_sc_gather = _make_sc_gather()


def kernel(states, action_categories, play_card_ids, attacking_card_ids,
           attacked_card_ids, evolving_card_ids, emb1, emb2, emb3,
           W1, b1, W2, b2):
    i32 = jnp.int32
    packed, base, t1, t2, t3a, t3d, t3e = _prep(
        action_categories.astype(i32), play_card_ids.astype(i32),
        attacking_card_ids.astype(i32), attacked_card_ids.astype(i32),
        evolving_card_ids.astype(i32),
        states, emb1, emb2, emb3, W1, b1.reshape(1, _MID), W2,
        b2.reshape(1, 1))
    out = _sc_gather(
        packed, base.reshape(-1),
        t1.reshape(-1), t2.reshape(-1),
        t3a.reshape(-1), t3d.reshape(-1), t3e.reshape(-1))
    return out.reshape(_B, 1)


# speedup vs baseline: 1.1274x; 1.1274x over previous
import functools

import jax
import jax.numpy as jnp
from jax import lax
from jax.experimental import pallas as pl
from jax.experimental.pallas import tpu as pltpu
from jax.experimental.pallas import tpu_sc as plsc

_B = 16384
_L = 20
_MID = 128
_NC = 2
_NS = 16
_NW = _NC * _NS
_BPW = _B // _NW
_GRP = _BPW // 16

_T1P, _T2P, _T3P = 128, 3072, 1024


def _prep_body(ac_ref, play_ref, atk_ref, dfd_ref, evo_ref, states_ref,
               emb1_ref, emb2_ref, emb3_ref, w1_ref, b1_ref, w2_ref, b2_ref,
               packed_ref, base_ref, t1_ref, t2_ref, t3a_ref, t3d_ref,
               t3e_ref):
    hi = lax.Precision.HIGHEST
    pieces = []
    for ref in (ac_ref, play_ref, atk_ref, dfd_ref, evo_ref):
        xt = jnp.transpose(ref[...].astype(jnp.float32))
        pieces.append(xt)
        pieces.append(jnp.zeros((4, _BPW), jnp.float32))
    pieces.append(jnp.zeros((8, _BPW), jnp.float32))
    packed_ref[0] = jnp.concatenate(pieces, axis=0).astype(jnp.int32)

    @pl.when(pl.program_id(0) == 0)
    def _():
        w2 = w2_ref[...]
        v = lax.dot_general(w2, w1_ref[...], (((1,), (0,)), ((), ())),
                            precision=hi)
        c = jnp.sum(b1_ref[...] * w2) + b2_ref[0, 0]

        def projT(vk, emb, pad):
            row = lax.dot_general(vk, emb, (((1,), (1,)), ((), ())),
                                  precision=hi)
            if pad:
                row = jnp.concatenate(
                    [row, jnp.zeros((1, pad), jnp.float32)], axis=1)
            return row

        base_ref[...] = projT(v[:, 0:128], states_ref[...], 0) + c
        t1_ref[...] = projT(v[:, 128:256], emb1_ref[...], _T1P - 5)
        t2_ref[...] = projT(v[:, 256:384], emb2_ref[...], _T2P - 3000)
        t3a_ref[...] = projT(v[:, 384:512], emb3_ref[...], _T3P - 1000)
        t3d_ref[...] = projT(v[:, 512:640], emb3_ref[...], _T3P - 1000)
        t3e_ref[...] = projT(v[:, 640:768], emb3_ref[...], _T3P - 1000)


_prep = pl.pallas_call(
    _prep_body,
    grid=(_NW,),
    in_specs=[
        pl.BlockSpec((_BPW, _L), lambda i: (i, 0)),
        pl.BlockSpec((_BPW, _L), lambda i: (i, 0)),
        pl.BlockSpec((_BPW, _L), lambda i: (i, 0)),
        pl.BlockSpec((_BPW, _L), lambda i: (i, 0)),
        pl.BlockSpec((_BPW, _L), lambda i: (i, 0)),
        pl.BlockSpec((_B, 128), lambda i: (0, 0)),
        pl.BlockSpec((5, 128), lambda i: (0, 0)),
        pl.BlockSpec((3000, 128), lambda i: (0, 0)),
        pl.BlockSpec((1000, 128), lambda i: (0, 0)),
        pl.BlockSpec((128, 768), lambda i: (0, 0)),
        pl.BlockSpec((1, 128), lambda i: (0, 0)),
        pl.BlockSpec((1, 128), lambda i: (0, 0)),
        pl.BlockSpec((1, 1), lambda i: (0, 0)),
    ],
    out_specs=[
        pl.BlockSpec((1, 128, _BPW), lambda i: (i, 0, 0)),
        pl.BlockSpec((1, _B), lambda i: (0, 0)),
        pl.BlockSpec((1, _T1P), lambda i: (0, 0)),
        pl.BlockSpec((1, _T2P), lambda i: (0, 0)),
        pl.BlockSpec((1, _T3P), lambda i: (0, 0)),
        pl.BlockSpec((1, _T3P), lambda i: (0, 0)),
        pl.BlockSpec((1, _T3P), lambda i: (0, 0)),
    ],
    out_shape=[
        jax.ShapeDtypeStruct((_NW, 128, _BPW), jnp.int32),
        jax.ShapeDtypeStruct((1, _B), jnp.float32),
        jax.ShapeDtypeStruct((1, _T1P), jnp.float32),
        jax.ShapeDtypeStruct((1, _T2P), jnp.float32),
        jax.ShapeDtypeStruct((1, _T3P), jnp.float32),
        jax.ShapeDtypeStruct((1, _T3P), jnp.float32),
        jax.ShapeDtypeStruct((1, _T3P), jnp.float32),
    ],
)


def _make_sc_gather():
    mesh = plsc.VectorSubcoreMesh(core_axis_name="c", subcore_axis_name="s")

    @functools.partial(
        pl.kernel,
        mesh=mesh,
        out_type=jax.ShapeDtypeStruct((_B,), jnp.float32),
        compiler_params=pltpu.CompilerParams(needs_layout_passes=False),
        scratch_types=[
            pltpu.VMEM((128, _BPW), jnp.int32),
            pltpu.VMEM((_T1P,), jnp.float32),
            pltpu.VMEM((_T2P,), jnp.float32),
            pltpu.VMEM((_T3P,), jnp.float32),
            pltpu.VMEM((_T3P,), jnp.float32),
            pltpu.VMEM((_T3P,), jnp.float32),
            pltpu.VMEM((_BPW,), jnp.float32),
            pltpu.VMEM((_BPW,), jnp.float32),
        ],
    )
    def sc_k(idx_hbm, base_hbm, t1_hbm, t2_hbm, t3a_hbm, t3d_hbm, t3e_hbm,
             out_hbm,
             tw, t1_v, t2_v, t3a_v, t3d_v, t3e_v, base_v, out_v):
        wid = lax.axis_index("s") * _NC + lax.axis_index("c")
        b0 = wid * _BPW
        pltpu.sync_copy(idx_hbm.at[wid], tw)
        pltpu.sync_copy(t1_hbm, t1_v)
        pltpu.sync_copy(t2_hbm, t2_v)
        pltpu.sync_copy(t3a_hbm, t3a_v)
        pltpu.sync_copy(t3d_hbm, t3d_v)
        pltpu.sync_copy(t3e_hbm, t3e_v)
        pltpu.sync_copy(base_hbm.at[pl.ds(b0, _BPW)], base_v)

        def group(gi, carry):
            s0 = gi * 16
            a1 = base_v[pl.ds(s0, 16)]
            a2 = a1 - a1
            a3 = a2
            a4 = a2
            a5 = a2
            for l in range(_L):
                a1 = a1 + plsc.load_gather(t1_v, [tw[l, pl.ds(s0, 16)]])
                a2 = a2 + plsc.load_gather(t2_v, [tw[24 + l, pl.ds(s0, 16)]])
                a3 = a3 + plsc.load_gather(
                    t3a_v, [tw[48 + l, pl.ds(s0, 16)]])
                a4 = a4 + plsc.load_gather(
                    t3d_v, [tw[72 + l, pl.ds(s0, 16)]])
                a5 = a5 + plsc.load_gather(
                    t3e_v, [tw[96 + l, pl.ds(s0, 16)]])
            out_v[pl.ds(s0, 16)] = (a1 + a2) + (a3 + a4) + a5
            return carry

        lax.fori_loop(0, _GRP, group, 0)
        pltpu.sync_copy(out_v, out_hbm.at[pl.ds(b0, _BPW)])

    return sc_k


_sc_gather = _make_sc_gather()


def kernel(states, action_categories, play_card_ids, attacking_card_ids,
           attacked_card_ids, evolving_card_ids, emb1, emb2, emb3,
           W1, b1, W2, b2):
    i32 = jnp.int32
    packed, base, t1, t2, t3a, t3d, t3e = _prep(
        action_categories.astype(i32), play_card_ids.astype(i32),
        attacking_card_ids.astype(i32), attacked_card_ids.astype(i32),
        evolving_card_ids.astype(i32),
        states, emb1, emb2, emb3, W1, b1.reshape(1, _MID), W2,
        b2.reshape(1, 1))
    out = _sc_gather(
        packed, base.reshape(-1),
        t1.reshape(-1), t2.reshape(-1),
        t3a.reshape(-1), t3d.reshape(-1), t3e.reshape(-1))
    return out.reshape(_B, 1)
